# baseline (speedup 1.0000x reference)
import functools

import jax
import jax.numpy as jnp
from jax import lax
from jax.experimental import pallas as pl
from jax.experimental.pallas import tpu as pltpu
from jax.experimental.pallas import tpu_sc as plsc

_B = 16384
_D = 64
_NC = 2
_NS = 16
_NW = _NC * _NS
_BPW = _B // _NW
_L = 16
_UNROLL = 8


def _mf_body(q_hbm, i_hbm, eq_hbm, ei_hbm, out_hbm,
             qidx_v, iidx_v, qrows_v, irows_v, scores_v, part_v, sem_q, sem_i):
    wid = lax.axis_index("s") * _NC + lax.axis_index("c")
    base = wid * _BPW

    pltpu.sync_copy(q_hbm.at[pl.ds(base, _BPW)], qidx_v)
    pltpu.sync_copy(i_hbm.at[pl.ds(base, _BPW)], iidx_v)
    cq = pltpu.async_copy(eq_hbm.at[qidx_v], qrows_v, sem_q)
    ci = pltpu.async_copy(ei_hbm.at[iidx_v], irows_v, sem_i)
    cq.wait()
    ci.wait()

    col0 = lax.iota(jnp.int32, _L) * _L

    def group(g, carry):
        r0 = g * _L
        for k in range(_L):
            r = r0 + k
            acc = qrows_v[r, pl.ds(0, _L)] * irows_v[r, pl.ds(0, _L)]
            for c in range(1, _D // _L):
                acc = acc + qrows_v[r, pl.ds(c * _L, _L)] * irows_v[r, pl.ds(c * _L, _L)]
            part_v[pl.ds(k * _L, _L)] = acc
        scores = plsc.load_gather(part_v, [col0])
        for l in range(1, _L):
            scores = scores + plsc.load_gather(part_v, [col0 + l])
        scores_v[pl.ds(r0, _L)] = 1.0 / (1.0 + jnp.exp(-scores))
        return carry

    lax.fori_loop(0, _BPW // _L, group, 0)

    pltpu.sync_copy(scores_v, out_hbm.at[pl.ds(base, _BPW)])


@jax.jit
def kernel(querys, items, embedding_query, embedding_item):
    mesh = plsc.VectorSubcoreMesh(
        core_axis_name="c", subcore_axis_name="s",
        num_cores=_NC, num_subcores=_NS)
    k = functools.partial(
        pl.kernel,
        out_type=jax.ShapeDtypeStruct((_B,), jnp.float32),
        mesh=mesh,
        compiler_params=pltpu.CompilerParams(
            needs_layout_passes=False, use_tc_tiling_on_sc=False),
        scratch_types=[
            pltpu.VMEM((_BPW,), jnp.int32),
            pltpu.VMEM((_BPW,), jnp.int32),
            pltpu.VMEM((_BPW, _D), jnp.float32),
            pltpu.VMEM((_BPW, _D), jnp.float32),
            pltpu.VMEM((_BPW,), jnp.float32),
            pltpu.VMEM((_L * _L,), jnp.float32),
            pltpu.SemaphoreType.DMA,
            pltpu.SemaphoreType.DMA,
        ],
    )(_mf_body)
    return k(querys, items, embedding_query, embedding_item)


# speedup vs baseline: 2.1679x; 2.1679x over previous
import functools

import jax
import jax.numpy as jnp
from jax import lax
from jax.experimental import pallas as pl
from jax.experimental.pallas import tpu as pltpu
from jax.experimental.pallas import tpu_sc as plsc

_B = 16384
_D = 64
_NC = 2
_NS = 16
_NW = _NC * _NS
_BPW = _B // _NW
_L = 16
_CH = 32
_NCHUNK = _BPW // _CH


def _scalar(vec, k):
    return jnp.squeeze(lax.slice(vec, (k,), (k + 1,)))


def _mf_body(q_hbm, i_hbm, eq_hbm, ei_hbm, out_hbm,
             qidx_v, iidx_v, qblk_v, iblk_v, part_v, scores_v, sem_q, sem_i):
    wid = lax.axis_index("s") * _NC + lax.axis_index("c")
    base = wid * _BPW

    pltpu.sync_copy(q_hbm.at[pl.ds(base, _BPW)], qidx_v)
    pltpu.sync_copy(i_hbm.at[pl.ds(base, _BPW)], iidx_v)

    col0 = lax.iota(jnp.int32, _L) * _L

    def chunk(c, carry):
        r0 = c * _CH
        qsubs = []
        isubs = []
        copies = []
        for g in range(_CH // _L):
            qv = qidx_v[pl.ds(r0 + g * _L, _L)]
            iv = iidx_v[pl.ds(r0 + g * _L, _L)]
            qb_vec = jnp.right_shift(qv, 3)
            ib_vec = jnp.right_shift(iv, 3)
            qsubs.append(jnp.bitwise_and(qv, 7))
            isubs.append(jnp.bitwise_and(iv, 7))
            for k in range(_L):
                kk = g * _L + k
                qb = _scalar(qb_vec, k)
                ib = _scalar(ib_vec, k)
                copies.append(pltpu.async_copy(eq_hbm.at[qb], qblk_v.at[kk], sem_q))
                copies.append(pltpu.async_copy(ei_hbm.at[ib], iblk_v.at[kk], sem_i))
        for cp in copies:
            cp.wait()
        for g in range(_CH // _L):
            for k in range(_L):
                kk = g * _L + k
                qs = _scalar(qsubs[g], k)
                us = _scalar(isubs[g], k)
                acc = qblk_v[kk, qs, pl.ds(0, _L)] * iblk_v[kk, us, pl.ds(0, _L)]
                for cc in range(1, _D // _L):
                    acc = acc + (qblk_v[kk, qs, pl.ds(cc * _L, _L)]
                                 * iblk_v[kk, us, pl.ds(cc * _L, _L)])
                part_v[pl.ds(k * _L, _L)] = acc
            scores = plsc.load_gather(part_v, [col0])
            for l in range(1, _L):
                scores = scores + plsc.load_gather(part_v, [col0 + l])
            scores_v[pl.ds(r0 + g * _L, _L)] = 1.0 / (1.0 + jnp.exp(-scores))
        return carry

    lax.fori_loop(0, _NCHUNK, chunk, 0)

    pltpu.sync_copy(scores_v, out_hbm.at[pl.ds(base, _BPW)])


@jax.jit
def kernel(querys, items, embedding_query, embedding_item):
    eq3 = embedding_query.reshape(1000000 // 8, 8, _D)
    ei3 = embedding_item.reshape(1000000 // 8, 8, _D)
    mesh = plsc.VectorSubcoreMesh(
        core_axis_name="c", subcore_axis_name="s",
        num_cores=_NC, num_subcores=_NS)
    k = functools.partial(
        pl.kernel,
        out_type=jax.ShapeDtypeStruct((_B,), jnp.float32),
        mesh=mesh,
        compiler_params=pltpu.CompilerParams(needs_layout_passes=False),
        scratch_types=[
            pltpu.VMEM((_BPW,), jnp.int32),
            pltpu.VMEM((_BPW,), jnp.int32),
            pltpu.VMEM((_CH, 8, _D), jnp.float32),
            pltpu.VMEM((_CH, 8, _D), jnp.float32),
            pltpu.VMEM((_L * _L,), jnp.float32),
            pltpu.VMEM((_BPW,), jnp.float32),
            pltpu.SemaphoreType.DMA,
            pltpu.SemaphoreType.DMA,
        ],
    )(_mf_body)
    return k(querys, items, eq3, ei3)
